# Initial kernel scaffold; baseline (speedup 1.0000x reference)
import jax
import jax.numpy as jnp
from jax.experimental import pallas as pl


def kernel(x, edge_index, W1a, b1a, W1b, b1b, W2a, b2a, W2b, b2b, Wl, bl):
    raise NotImplementedError("write your pallas kernel here")


# speedup vs baseline: 3.7112x; 3.7112x over previous
import functools

import jax
import jax.numpy as jnp
from jax import lax
from jax.experimental import pallas as pl
from jax.experimental.pallas import tpu as pltpu
from jax.experimental.pallas import tpu_sc as plsc

N_NODES = 10000
N_EDGES = 160000
D = 256
H = 128

NS = 16
CHUNK = 128
NCH = -(-N_EDGES // (NS * CHUNK))
EPT = NCH * CHUNK
E_PAD = EPT * NS
ROWS_PT = N_NODES // NS
ACC_ROWS = N_NODES + 16

_sc_mesh = plsc.VectorSubcoreMesh(core_axis_name="c", subcore_axis_name="s")


@functools.partial(
    pl.kernel,
    out_type=jax.ShapeDtypeStruct((N_NODES, 2, H), jnp.float32),
    mesh=_sc_mesh,
    scratch_types=[
        pltpu.VMEM((NCH, CHUNK), jnp.int32),
        pltpu.VMEM((NCH, CHUNK), jnp.int32),
        pltpu.VMEM((CHUNK,), jnp.int32),
        pltpu.VMEM((CHUNK, H), jnp.float32),
        pltpu.VMEM_SHARED((ACC_ROWS, H), jnp.float32),
        pltpu.SemaphoreType.DMA,
    ],
)
def _sc_aggregate(x2_hbm, x3_hbm, src_hbm, dst_hbm, out_hbm,
                  srcb, dstb, src_v, rows_v, acc, sem):
    c = lax.axis_index("c")
    s = lax.axis_index("s")
    r0 = s * ROWS_PT
    pltpu.sync_copy(src_hbm.at[s], srcb)
    pltpu.sync_copy(dst_hbm.at[s], dstb)
    pltpu.sync_copy(x3_hbm.at[pl.ds(r0, ROWS_PT), c], acc.at[pl.ds(r0, ROWS_PT)])
    plsc.subcore_barrier()

    def body(j, carry):
        for i in range(CHUNK // 16):
            sl = pl.ds(i * 16, 16)
            v = srcb[j, sl]
            src_v[sl] = v + v + c
        pltpu.async_copy(x2_hbm.at[src_v], rows_v, sem).wait()
        pltpu.sync_copy(rows_v, acc.at[dstb.at[j]], add=True)
        return carry

    lax.fori_loop(0, NCH, body, 0)
    plsc.subcore_barrier()
    pltpu.sync_copy(acc.at[pl.ds(r0, ROWS_PT)], out_hbm.at[pl.ds(r0, ROWS_PT), c])


def _mlp2_body(g_ref, wa_ref, ba_ref, wb_ref, bb_ref, o_ref):
    h = jnp.dot(g_ref[...], wa_ref[...], preferred_element_type=jnp.float32)
    h = jnp.maximum(h + ba_ref[...], 0.0)
    h = jnp.dot(h, wb_ref[...], preferred_element_type=jnp.float32)
    o_ref[...] = jnp.maximum(h + bb_ref[...], 0.0)


def _mlp3_body(g_ref, wa_ref, ba_ref, wb_ref, bb_ref, wl_ref, bl_ref, o_ref):
    h = jnp.dot(g_ref[...], wa_ref[...], preferred_element_type=jnp.float32)
    h = jnp.maximum(h + ba_ref[...], 0.0)
    h = jnp.dot(h, wb_ref[...], preferred_element_type=jnp.float32)
    h = jnp.maximum(h + bb_ref[...], 0.0)
    o_ref[...] = jnp.dot(h, wl_ref[...], preferred_element_type=jnp.float32) + bl_ref[...]


_ROW_BLK = 1000
_row_spec = pl.BlockSpec((_ROW_BLK, D), lambda i: (i, 0))
_w_spec = pl.BlockSpec((D, D), lambda i: (0, 0))
_b_spec = pl.BlockSpec((1, D), lambda i: (0, 0))


def _mlp2(g, wa, ba, wb, bb):
    return pl.pallas_call(
        _mlp2_body,
        grid=(N_NODES // _ROW_BLK,),
        in_specs=[_row_spec, _w_spec, _b_spec, _w_spec, _b_spec],
        out_specs=_row_spec,
        out_shape=jax.ShapeDtypeStruct((N_NODES, D), jnp.float32),
    )(g, wa, ba, wb, bb)


def _mlp3(g, wa, ba, wb, bb, wl, bl):
    return pl.pallas_call(
        _mlp3_body,
        grid=(N_NODES // _ROW_BLK,),
        in_specs=[_row_spec, _w_spec, _b_spec, _w_spec, _b_spec, _w_spec, _b_spec],
        out_specs=_row_spec,
        out_shape=jax.ShapeDtypeStruct((N_NODES, D), jnp.float32),
    )(g, wa, ba, wb, bb, wl, bl)


def kernel(x, edge_index, W1a, b1a, W1b, b1b, W2a, b2a, W2b, b2b, Wl, bl):
    src = edge_index[0].astype(jnp.int32)
    dst = edge_index[1].astype(jnp.int32)
    npad = E_PAD - N_EDGES
    src_p = jnp.concatenate([src, jnp.zeros((npad,), jnp.int32)])
    dst_p = jnp.concatenate(
        [dst, N_NODES + (jnp.arange(npad, dtype=jnp.int32) & 15)])
    src_p = src_p.reshape(NS, NCH, CHUNK)
    dst_p = dst_p.reshape(NS, NCH, CHUNK)

    ba1, bb1 = b1a.reshape(1, D), b1b.reshape(1, D)
    ba2, bb2 = b2a.reshape(1, D), b2b.reshape(1, D)
    blr = bl.reshape(1, D)

    g1 = _sc_aggregate(x.reshape(2 * N_NODES, H), x.reshape(N_NODES, 2, H),
                       src_p, dst_p)
    h1 = _mlp2(g1.reshape(N_NODES, D), W1a, ba1, W1b, bb1)
    g2 = _sc_aggregate(h1.reshape(2 * N_NODES, H), h1.reshape(N_NODES, 2, H),
                       src_p, dst_p)
    out = _mlp3(g2.reshape(N_NODES, D), W2a, ba2, W2b, bb2, Wl, blr)
    return out
